# baseline (speedup 1.0000x reference)
import jax
import jax.numpy as jnp
from jax import lax
from jax.experimental import pallas as pl
from jax.experimental.pallas import tpu as pltpu

_K = 1024
_D = 256
_B = 8
_N = 576
_BETA = 0.1
_ROWS = _B * _N
_BR = 4608
_NBLK = _ROWS // _BR
_LSCALE = _BETA / (_ROWS * _D)


def _vq_tc_body(zb_ref, e_ref, codes_ref, zq_ref, lacc_ref, c_ref):
    i = pl.program_id(0)

    @pl.when(i == 0)
    def _():
        e = e_ref[...]
        c_ref[...] = jnp.sum(e * e, axis=1).reshape(1, _K)
        lacc_ref[...] = jnp.zeros((1, 1), jnp.float32)

    zb = zb_ref[...]
    zb2 = zb * -2.0
    mm2 = lax.dot_general(
        zb2, e_ref[...],
        dimension_numbers=(((1,), (1,)), ((), ())),
        preferred_element_type=jnp.float32,
    )
    a = jnp.sum(zb2 * zb2, axis=1, keepdims=True) * 0.25
    dist = a + mm2 + c_ref[...]
    minv = jnp.min(dist, axis=1)
    iota_f = lax.broadcasted_iota(jnp.int32, (_BR, _K), 1).astype(jnp.float32)
    sel = jnp.where(dist == minv[:, None], iota_f, float(_K))
    code_f = jnp.min(sel, axis=1)
    codes_ref[0, 0, :] = code_f.astype(jnp.int32)
    onehot = jnp.where(sel == code_f[:, None], 1.0, 0.0)
    zq_ref[...] = lax.dot_general(
        onehot, e_ref[...],
        dimension_numbers=(((1,), (0,)), ((), ())),
        preferred_element_type=jnp.float32,
    )

    lacc_ref[...] += jnp.sum(minv).reshape(1, 1)

    @pl.when(i == _NBLK - 1)
    def _():
        lacc_ref[...] = lacc_ref[...] * _LSCALE


def _tc_call(flat_z, e):
    return pl.pallas_call(
        _vq_tc_body,
        grid=(_NBLK,),
        in_specs=[
            pl.BlockSpec((_BR, _D), lambda i: (i, 0)),
            pl.BlockSpec((_K, _D), lambda i: (0, 0)),
        ],
        out_specs=[
            pl.BlockSpec((1, 1, _BR), lambda i: (i, 0, 0)),
            pl.BlockSpec((_BR, _D), lambda i: (i, 0)),
            pl.BlockSpec((1, 1), lambda i: (0, 0)),
        ],
        out_shape=[
            jax.ShapeDtypeStruct((_NBLK, 1, _BR), jnp.int32),
            jax.ShapeDtypeStruct((_ROWS, _D), jnp.float32),
            jax.ShapeDtypeStruct((1, 1), jnp.float32),
        ],
        scratch_shapes=[pltpu.VMEM((1, _K), jnp.float32)],
    )(flat_z, e)


def kernel(z, embed_weight):
    b, n, d = z.shape
    flat_z = z.reshape(-1, d)
    codes3d, z_q, lacc = _tc_call(flat_z, embed_weight)
    return z_q.reshape(b, n, d), codes3d.reshape(b, n), lacc[0, 0]


# speedup vs baseline: 1.1053x; 1.1053x over previous
import jax
import jax.numpy as jnp
from jax import lax
from jax.experimental import pallas as pl
from jax.experimental.pallas import tpu as pltpu

_K = 1024
_D = 256
_B = 8
_N = 576
_BETA = 0.1
_ROWS = _B * _N
_BR = 2304
_NBLK = _ROWS // _BR
_LSCALE = _BETA / (_ROWS * _D)


def _vq_tc_body(zb_ref, e_ref, codes_ref, zq_ref, lacc_ref, c_ref):
    i = pl.program_id(0)

    @pl.when(i == 0)
    def _():
        e = e_ref[...]
        c_ref[...] = jnp.sum(e * e, axis=1).reshape(1, _K)
        lacc_ref[...] = jnp.zeros((1, 1), jnp.float32)

    zb = zb_ref[...]
    zb2 = zb * -2.0
    mm2 = lax.dot_general(
        zb2, e_ref[...],
        dimension_numbers=(((1,), (1,)), ((), ())),
        preferred_element_type=jnp.float32,
    )
    a = jnp.sum(zb2 * zb2, axis=1, keepdims=True) * 0.25
    dist = a + mm2 + c_ref[...]
    minv = jnp.min(dist, axis=1)
    iota_f = lax.broadcasted_iota(jnp.int32, (_BR, _K), 1).astype(jnp.float32)
    sel = jnp.where(dist == minv[:, None], iota_f, float(_K))
    code_f = jnp.min(sel, axis=1)
    code_i = code_f.astype(jnp.int32)
    for r in range(4):
        codes_ref[pl.ds(4 * i + r, 1), :] = code_i[r * 576:(r + 1) * 576][None, :]
    onehot = jnp.where(sel == code_f[:, None], 1.0, 0.0)
    zq_ref[...] = lax.dot_general(
        onehot, e_ref[...],
        dimension_numbers=(((1,), (0,)), ((), ())),
        preferred_element_type=jnp.float32,
    )

    lacc_ref[...] += jnp.sum(minv).reshape(1, 1)

    @pl.when(i == _NBLK - 1)
    def _():
        lacc_ref[...] = lacc_ref[...] * _LSCALE


def _tc_call(flat_z, e):
    return pl.pallas_call(
        _vq_tc_body,
        grid=(_NBLK,),
        in_specs=[
            pl.BlockSpec((_BR, _D), lambda i: (i, 0)),
            pl.BlockSpec((_K, _D), lambda i: (0, 0)),
        ],
        out_specs=[
            pl.BlockSpec((_B, _N), lambda i: (0, 0)),
            pl.BlockSpec((_BR, _D), lambda i: (i, 0)),
            pl.BlockSpec((1, 1), lambda i: (0, 0)),
        ],
        out_shape=[
            jax.ShapeDtypeStruct((_B, _N), jnp.int32),
            jax.ShapeDtypeStruct((_ROWS, _D), jnp.float32),
            jax.ShapeDtypeStruct((1, 1), jnp.float32),
        ],
        scratch_shapes=[pltpu.VMEM((1, _K), jnp.float32)],
    )(flat_z, e)


def kernel(z, embed_weight):
    b, n, d = z.shape
    flat_z = z.reshape(-1, d)
    codes, z_q, lacc = _tc_call(flat_z, embed_weight)
    return z_q.reshape(b, n, d), codes, lacc[0, 0]
